# baseline (speedup 1.0000x reference)
import functools

import jax
import jax.numpy as jnp
from jax import lax
from jax.experimental import pallas as pl
from jax.experimental.pallas import tpu as pltpu
from jax.experimental.pallas import tpu_sc as plsc

BATCH = 16384
VOCAB = 1000000
EMBED_DIM = 16
LANES = 128
PACK = LANES // EMBED_DIM

_NW = 32
_BPW = BATCH // _NW
_CHUNK = 128
_NCHUNK = _BPW // _CHUNK
_GROUPS = _CHUNK // 16


def _make_lookup():
    mesh = plsc.VectorSubcoreMesh(core_axis_name="c", subcore_axis_name="s")

    @functools.partial(
        pl.kernel,
        mesh=mesh,
        out_type=jax.ShapeDtypeStruct((BATCH * EMBED_DIM,), jnp.float32),
        scratch_types=[
            pltpu.VMEM((_NCHUNK, _CHUNK), jnp.int32),
            pltpu.VMEM((_NCHUNK, _CHUNK), jnp.int32),
            pltpu.VMEM((_NCHUNK, _CHUNK, LANES), jnp.float32),
            pltpu.VMEM((_NCHUNK, _CHUNK * EMBED_DIM), jnp.float32),
            pltpu.SemaphoreType.DMA,
            pltpu.SemaphoreType.DMA,
        ],
        compiler_params=pltpu.CompilerParams(needs_layout_passes=False),
    )
    def lookup(idx_hbm, table_hbm, out_hbm, idx_v, pidx_v, rows_v, out_v,
               gsem, osem):
        wid = lax.axis_index("s") * 2 + lax.axis_index("c")
        base = wid * _BPW
        lane = lax.iota(jnp.int32, 16)
        for j in range(_NCHUNK):
            pltpu.sync_copy(idx_hbm.at[pl.ds(base + j * _CHUNK, _CHUNK)],
                            idx_v.at[j])
            for g in range(_GROUPS):
                v = idx_v[j, pl.ds(g * 16, 16)]
                pidx_v[j, pl.ds(g * 16, 16)] = lax.shift_right_logical(v, 3)
        copies = [
            pltpu.async_copy(table_hbm.at[pidx_v.at[j]], rows_v.at[j], gsem)
            for j in range(_NCHUNK)
        ]
        out_copies = []
        for j in range(_NCHUNK):
            copies[j].wait()
            jsplat = jnp.full((16,), j, dtype=jnp.int32)

            def extract_group(g, carry, j=j, jsplat=jsplat):
                jrel = lane + g * 16
                ivec = idx_v[j, pl.ds(pl.multiple_of(g * 16, 16), 16)]
                colbase = lax.shift_left(
                    jnp.bitwise_and(ivec, jnp.int32(PACK - 1)), 4)
                rowbase = lax.shift_left(jrel, 4)
                for d in range(EMBED_DIM):
                    vals = plsc.load_gather(rows_v, [jsplat, jrel, colbase + d])
                    plsc.store_scatter(out_v, [jsplat, rowbase + d], vals)
                return carry

            lax.fori_loop(0, _GROUPS, extract_group, 0)
            out_copies.append(
                pltpu.async_copy(
                    out_v.at[j],
                    out_hbm.at[pl.ds((base + j * _CHUNK) * EMBED_DIM,
                                     _CHUNK * EMBED_DIM)], osem))
        for c in out_copies:
            c.wait()

    return lookup


_lookup = _make_lookup()


@jax.jit
def kernel(inputs, embedding_table):
    input_shape = inputs.shape
    flat = jnp.reshape(inputs, (-1,)).astype(jnp.int32)
    packed_table = jnp.reshape(embedding_table, (VOCAB // PACK, LANES))
    out = _lookup(flat, packed_table)
    return jnp.reshape(out, input_shape + (EMBED_DIM,))


# speedup vs baseline: 5.9211x; 5.9211x over previous
import functools

import jax
import jax.numpy as jnp
from jax import lax
from jax.experimental import pallas as pl
from jax.experimental.pallas import tpu as pltpu
from jax.experimental.pallas import tpu_sc as plsc

BATCH = 16384
VOCAB = 1000000
EMBED_DIM = 16
LANES = 128

_NW = 32
_BPW = BATCH // _NW
_W = 16
_NWAVES = _BPW // _W


def _make_lookup():
    mesh = plsc.VectorSubcoreMesh(core_axis_name="c", subcore_axis_name="s")

    @functools.partial(
        pl.kernel,
        mesh=mesh,
        out_type=jax.ShapeDtypeStruct((EMBED_DIM, BATCH), jnp.float32),
        scratch_types=[
            pltpu.VMEM((_BPW,), jnp.int32),
            pltpu.VMEM((2, _W, EMBED_DIM, LANES), jnp.float32),
            pltpu.VMEM((EMBED_DIM, _BPW), jnp.float32),
            pltpu.SemaphoreType.DMA,
            pltpu.SemaphoreType.DMA,
            pltpu.SemaphoreType.DMA,
        ],
        compiler_params=pltpu.CompilerParams(needs_layout_passes=False),
    )
    def lookup(idx_hbm, table_hbm, out_hbm, idx_v, rows_v, obuf,
               sem0, sem1, osem):
        wid = lax.axis_index("s") * 2 + lax.axis_index("c")
        base = wid * _BPW
        lane = lax.iota(jnp.int32, 16)
        pltpu.sync_copy(idx_hbm.at[pl.ds(base, _BPW)], idx_v)
        sems = [sem0, sem1]

        def fire(w, parity):
            sem = sems[0] if parity == 0 else sems[1]
            offv = jnp.bitwise_and(idx_v[pl.ds(w * _W, _W)], jnp.int32(-LANES))
            for k in range(_W):
                off_s = jnp.max(jnp.where(lane == k, offv, jnp.int32(0)))
                off = pl.multiple_of(off_s, LANES)
                pltpu.async_copy(
                    table_hbm.at[pl.ds(0, 8), pl.ds(off, LANES)],
                    rows_v.at[parity, k, pl.ds(0, 8)], sem)
                pltpu.async_copy(
                    table_hbm.at[pl.ds(8, 8), pl.ds(off, LANES)],
                    rows_v.at[parity, k, pl.ds(8, 8)], sem)

        def drain(parity):
            sem = sems[0] if parity == 0 else sems[1]
            for k in range(_W):
                pltpu.make_async_copy(table_hbm.at[:, pl.ds(0, LANES)],
                                      rows_v.at[parity, k], sem).wait()

        def run_wave(w, parity):
            @pl.when(w + 1 < _NWAVES)
            def _():
                fire(w + 1, 1 - parity)

            drain(parity)
            psplat = jnp.full((16,), parity, dtype=jnp.int32)
            for k in range(_W):
                b = w * _W + k
                bsplat = lax.broadcast(b, (16,))
                lsplat = jnp.bitwise_and(
                    plsc.load_gather(idx_v, [bsplat]), jnp.int32(LANES - 1))
                ksplat = jnp.full((16,), k, dtype=jnp.int32)
                vals = plsc.load_gather(rows_v, [psplat, ksplat, lane, lsplat])
                plsc.store_scatter(obuf, [lane, bsplat], vals)

        fire(0, 0)

        def body(w, carry):
            run_wave(2 * w, 0)
            run_wave(2 * w + 1, 1)
            return carry

        lax.fori_loop(0, _NWAVES // 2, body, 0)
        pltpu.async_copy(obuf, out_hbm.at[:, pl.ds(base, _BPW)], osem).wait()

    return lookup


_lookup = _make_lookup()


@jax.jit
def kernel(inputs, embedding_table):
    input_shape = inputs.shape
    flat = jnp.reshape(inputs, (-1,)).astype(jnp.int32)
    table_t = jnp.transpose(embedding_table)
    out_t = _lookup(flat, table_t)
    return jnp.reshape(jnp.transpose(out_t), input_shape + (EMBED_DIM,))
